# baseline (speedup 1.0000x reference)
import functools

import jax
import jax.numpy as jnp
from jax import lax
from jax.experimental import pallas as pl
from jax.experimental.pallas import tpu as pltpu
from jax.experimental.pallas import tpu_sc as plsc

N = 100000
NP = 100352
C = NP // 16
VR = C // 16
T = 1 << 17
TM = T - 1
OT = T + 2048
R = 4096
NRANGE = T // R
NBLK = 2048
SPPROWS = 199 * NBLK

_NC, _NS = 2, 16
_mesh = plsc.VectorSubcoreMesh(core_axis_name="c", subcore_axis_name="s",
                               num_cores=_NC, num_subcores=_NS)
_sc_params = pltpu.CompilerParams(needs_layout_passes=False)


def _phase_a_body(x_ref, wred_ref, bred_ref, wcat_ref, rf_ref, g_ref):
    x = x_ref[...]
    rf = jnp.maximum(
        jnp.dot(x, wred_ref[...], preferred_element_type=jnp.float32)
        + bred_ref[...], 0.0)
    rf_ref[...] = rf
    rfw = jnp.dot(rf, wcat_ref[...], preferred_element_type=jnp.float32)
    g_ref[0] = rfw[:, 0:128]
    g_ref[1] = rfw[:, 128:256]


def _phase_a(x, w_red, b_red, w_cat):
    n, cin = x.shape
    cr = w_red.shape[1]
    return pl.pallas_call(
        _phase_a_body,
        grid=(n // NBLK,),
        in_specs=[
            pl.BlockSpec((NBLK, cin), lambda i: (i, 0)),
            pl.BlockSpec((cin, cr), lambda i: (0, 0)),
            pl.BlockSpec((1, cr), lambda i: (0, 0)),
            pl.BlockSpec((cr, 256), lambda i: (0, 0)),
        ],
        out_specs=[
            pl.BlockSpec((NBLK, cr), lambda i: (i, 0)),
            pl.BlockSpec((2, NBLK, 128), lambda i: (0, i, 0)),
        ],
        out_shape=[
            jax.ShapeDtypeStruct((n, cr), jnp.float32),
            jax.ShapeDtypeStruct((2, n, 128), jnp.float32),
        ],
    )(x, w_red, b_red.reshape(1, cr), w_cat)


def _phase_b_body(m0_ref, m1_ref, m2_ref, m3_ref,
                  rf_ref, bfcl_ref, wfc_ref, wfcs_ref,
                  bfcs_ref, wout_ref, wlo1_ref, wlo2_ref, blo2_ref, out_ref):
    atts = []
    for j, mr in enumerate((m0_ref, m1_ref, m2_ref, m3_ref)):
        h = (j % 2) * 64
        co = 64 - h
        cj = jnp.maximum(mr[:, co:co + 1], 1.0)
        atts.append(jnp.maximum(
            mr[:, h:h + 64] / cj + bfcl_ref[:, 64 * j:64 * (j + 1)], 0.0))
    a0, a1, a2, a3 = atts
    feat_s = a0 + a1 + a2 + a3
    feat_z = jnp.maximum(
        jnp.dot(feat_s, wfc_ref[...], preferred_element_type=jnp.float32), 0.0)
    av = jax.nn.sigmoid(
        jnp.dot(feat_z, wfcs_ref[...], preferred_element_type=jnp.float32)
        + bfcs_ref[...])
    sf = (a0 * av[:, 0:64] + a1 * av[:, 64:128]
          + a2 * av[:, 128:192] + a3 * av[:, 192:256])
    out128 = jnp.dot(sf, wout_ref[...], preferred_element_type=jnp.float32)
    rf = rf_ref[...]
    hh = jnp.maximum(
        jnp.dot(rf, wlo1_ref[:128, :], preferred_element_type=jnp.float32)
        + jnp.dot(out128, wlo1_ref[128:, :],
                  preferred_element_type=jnp.float32), 0.0)
    out_ref[...] = (jnp.dot(hh, wlo2_ref[...],
                            preferred_element_type=jnp.float32) + blo2_ref[...])


def _phase_b(spp, rf, b_fcl_cat, w_fc, w_fcs_cat, b_fcs_cat, w_out,
             w_lo1, w_lo2, b_lo2):
    n = rf.shape[0]
    nb = n // NBLK
    cout = w_lo2.shape[1]
    mean_specs = [pl.BlockSpec((NBLK, 128), lambda i, j=j: (j * nb + i, 0))
                  for j in range(4)]
    return pl.pallas_call(
        _phase_b_body,
        grid=(nb,),
        in_specs=mean_specs + [
            pl.BlockSpec((NBLK, 128), lambda i: (i, 0)),
            pl.BlockSpec((1, 256), lambda i: (0, 0)),
            pl.BlockSpec((64, 64), lambda i: (0, 0)),
            pl.BlockSpec((64, 256), lambda i: (0, 0)),
            pl.BlockSpec((1, 256), lambda i: (0, 0)),
            pl.BlockSpec((64, 128), lambda i: (0, 0)),
            pl.BlockSpec((256, 128), lambda i: (0, 0)),
            pl.BlockSpec((128, 256), lambda i: (0, 0)),
            pl.BlockSpec((1, 256), lambda i: (0, 0)),
        ],
        out_specs=pl.BlockSpec((NBLK, cout), lambda i: (i, 0)),
        out_shape=jax.ShapeDtypeStruct((n, cout), jnp.float32),
    )(spp, spp, spp, spp, rf,
      b_fcl_cat.reshape(1, 256), w_fc, w_fcs_cat,
      b_fcs_cat.reshape(1, 256), w_out, w_lo1, w_lo2,
      b_lo2.reshape(1, -1))


@functools.partial(
    pl.kernel, mesh=_mesh, compiler_params=_sc_params,
    out_type=jax.ShapeDtypeStruct((SPPROWS, 128), jnp.float32),
    scratch_types=[
        pltpu.VMEM((16,), jnp.int32),
        pltpu.VMEM((C + 16,), jnp.int32),
        pltpu.VMEM((C + 16,), jnp.int32),
        pltpu.VMEM((C + 16,), jnp.int32),
        pltpu.VMEM((C + 128,), jnp.int32),
        pltpu.VMEM((C + 128,), jnp.int32),
        pltpu.VMEM((C + 128,), jnp.int32),
        pltpu.VMEM((C + 128,), jnp.int32),
        pltpu.VMEM((C + 128,), jnp.int32),
        pltpu.VMEM((C + 128,), jnp.int32),
        pltpu.VMEM((C + 128,), jnp.int32),
        pltpu.VMEM((256,), jnp.int32),
        pltpu.VMEM((64, 128), jnp.float32),
        pltpu.VMEM((64, 128), jnp.float32),
        pltpu.VMEM_SHARED((OT,), jnp.int32),
        pltpu.VMEM_SHARED((R + 16, 128), jnp.float32),
        pltpu.VMEM_SHARED((256,), jnp.int32),
        pltpu.SemaphoreType.DMA,
        pltpu.SemaphoreType.DMA,
    ])
def _segment_kernel(coords_t, params_c, iota_c, neg1_c, zero_c,
                    dmy_c, zrow_c, gflat, spp_out,
                    par_v, kh_c, kl_c, slot_c, live_v,
                    slots_l, owners_l, okh_l, okl_l,
                    clm_s, clm_v, statsb, gbuf, zv,
                    owner_sh, sums_sh, stats_sh, sem, semg):
    core = lax.axis_index("c")
    s = lax.axis_index("s")
    w0 = s * C
    lane = lax.iota(jnp.int32, 16)

    pltpu.sync_copy(params_c, par_v)
    pltpu.sync_copy(zrow_c.at[pl.ds(0, 64), :], zv)
    pv = par_v[...]

    def extract(vec, i):
        return jnp.sum(jnp.where(lane == i, vec, 0))

    def last(vec):
        return jnp.sum(jnp.where(lane == 15, vec, 0))

    pltpu.sync_copy(iota_c, slots_l)
    pltpu.sync_copy(dmy_c, clm_s)
    pltpu.sync_copy(zero_c, clm_v)

    def scale_body(t, _unused):
        sidx = core * 2 + t
        q = extract(pv, sidx)
        mg = extract(pv, sidx + 4)

        for r in range(2):
            pltpu.sync_copy(coords_t.at[pl.ds(r * NP + w0, C)],
                            okl_l.at[pl.ds(0, C)])

            def keypass(v, _, r=r):
                ab = okl_l[pl.ds(v * 16, 16)]
                hi = lax.shift_right_logical(ab, 9)
                lo = ab & 511
                if r == 0:
                    kh_c[pl.ds(v * 16, 16)] = (
                        hi * q + lax.shift_right_logical(lo * mg, 14))
                else:
                    kl_c[pl.ds(v * 16, 16)] = (
                        lax.shift_right_logical(hi * mg, 14) * q
                        + lax.shift_right_logical(lo * mg, 14))
                return 0
            lax.fori_loop(0, VR, keypass, 0)

        def hashpass(v, _):
            kh = kh_c[pl.ds(v * 16, 16)]
            kl = kl_c[pl.ds(v * 16, 16)]
            u = kh * (-1640531527) + kl * (-2048144777)
            u = u ^ lax.shift_right_logical(u, 15)
            u = u * (-1028477379)
            u = u ^ lax.shift_right_logical(u, 13)
            slot_c[pl.ds(v * 16, 16)] = u & TM
            return 0
        lax.fori_loop(0, VR, hashpass, 0)

        ob = s * (OT // 16)
        pltpu.sync_copy(neg1_c, owner_sh.at[pl.ds(ob, 6400)])
        pltpu.sync_copy(neg1_c.at[pl.ds(0, 1920)],
                        owner_sh.at[pl.ds(ob + 6400, 1920)])
        pltpu.sync_copy(iota_c, live_v)
        plsc.subcore_barrier()

        def cond(carry):
            return carry[0] > 0

        def round_body(carry):
            _, live_n = carry
            nv = (live_n + 15) // 16

            def p1(v, _):
                li = live_v[pl.ds(v * 16, 16)]
                slots_l[pl.ds(v * 16, 16)] = plsc.load_gather(slot_c, [li])
                return 0
            lax.fori_loop(0, nv, p1, 0)

            def gathers(ln):
                pltpu.async_copy(owner_sh.at[slots_l.at[pl.ds(0, ln)]],
                                 owners_l.at[pl.ds(0, ln)], sem).wait()

                def p2(v, _):
                    o = owners_l[pl.ds(v * 16, 16)]
                    slots_l[pl.ds(v * 16, 16)] = jnp.maximum(o, 0)
                    return 0
                lax.fori_loop(0, nv, p2, 0)

                pltpu.async_copy(
                    coords_t.at[slots_l.at[pl.ds(0, ln)]],
                    okh_l.at[pl.ds(0, ln)], sem).wait()

                def p2b(v, _):
                    slots_l[pl.ds(v * 16, 16)] = (
                        slots_l[pl.ds(v * 16, 16)] + NP)
                    return 0
                lax.fori_loop(0, nv, p2b, 0)
                pltpu.async_copy(
                    coords_t.at[slots_l.at[pl.ds(0, ln)]],
                    okl_l.at[pl.ds(0, ln)], semg).wait()

            big = live_n > 1520
            mid = live_n > 368

            @pl.when(big)
            def _():
                gathers(C + 128)

            @pl.when(jnp.logical_and(~big, mid))
            def _():
                gathers(1536)

            @pl.when(~mid)
            def _():
                gathers(384)

            def p3(v, carry2):
                off_k, off_e = carry2
                gi = v * 16 + lane
                valid = gi < live_n
                li = live_v[pl.ds(v * 16, 16)]
                slots = plsc.load_gather(slot_c, [li])
                myh = plsc.load_gather(kh_c, [li])
                myl = plsc.load_gather(kl_c, [li])
                uu = myh * (-1640531527) + myl * (-2048144777)
                uu = uu ^ lax.shift_right_logical(uu, 15)
                uu = uu * (-1028477379)
                uu = uu ^ lax.shift_right_logical(uu, 13)
                steps = (lax.shift_right_logical(uu, 17) | 1) & TM
                o = owners_l[pl.ds(v * 16, 16)]
                oa = okh_l[pl.ds(v * 16, 16)]
                ob2 = okl_l[pl.ds(v * 16, 16)]
                okh = (lax.shift_right_logical(oa, 9) * q
                       + lax.shift_right_logical((oa & 511) * mg, 14))
                okl = (lax.shift_right_logical(
                           lax.shift_right_logical(ob2, 9) * mg, 14) * q
                       + lax.shift_right_logical((ob2 & 511) * mg, 14))
                claimed = o >= 0
                match = valid & claimed & (okh == myh) & (okl == myl)
                adv = valid & claimed & (~match)
                plsc.store_scatter(slot_c, [jnp.where(adv, li, C)],
                                   (slots + steps) & TM)
                keep = valid & (~match)
                cum_k = plsc.cumsum(jnp.where(keep, 1, 0))
                pos_k = jnp.where(keep, off_k + cum_k - 1, C + 64)
                plsc.store_scatter(live_v, [pos_k], li)
                empty = valid & (~claimed)
                cum_e = plsc.cumsum(jnp.where(empty, 1, 0))
                pos_e = jnp.where(empty, off_e + cum_e - 1, C + 64)
                plsc.store_scatter(clm_s, [pos_e], slots)
                plsc.store_scatter(clm_v, [pos_e], w0 + li)
                return off_k + last(cum_k), off_e + last(cum_e)
            new_live, n_claim = lax.fori_loop(0, nv, p3, (0, 0))

            def padclaims(k, _):
                pos = k * 16 + lane
                m = pos >= n_claim
                tgt = jnp.where(m, pos, C + 64)
                plsc.store_scatter(clm_s, [tgt], T + (pos & 2047))
                plsc.store_scatter(clm_v, [tgt], jnp.zeros((16,), jnp.int32))
                return 0

            def claimscat(ln):
                lax.fori_loop(n_claim // 16, ln // 16, padclaims, 0)
                pltpu.sync_copy(clm_v.at[pl.ds(0, ln)],
                                owner_sh.at[clm_s.at[pl.ds(0, ln)]])

            @pl.when(big)
            def _():
                claimscat(C + 128)

            @pl.when(jnp.logical_and(~big, mid))
            def _():
                claimscat(1536)

            @pl.when(~mid)
            def _():
                claimscat(384)

            statsb[pl.ds(0, 16)] = jnp.full((16,), new_live, jnp.int32)
            pltpu.sync_copy(statsb.at[pl.ds(0, 16)],
                            stats_sh.at[pl.ds(s * 16, 16)])
            plsc.subcore_barrier()
            pltpu.sync_copy(stats_sh, statsb)

            def sumstats(v, acc):
                return acc + jnp.sum(statsb[pl.ds(v * 16, 16)])
            total = lax.fori_loop(0, 16, sumstats, 0)
            return total, new_live

        lax.while_loop(cond, round_body, (jnp.int32(1), jnp.int32(C)))

        def range_body(rho, _):
            base = rho * R
            def zk(k, _):
                pltpu.sync_copy(zv,
                                sums_sh.at[pl.ds(s * 256 + k * 64, 64), :])
                return 0
            lax.fori_loop(0, 4, zk, 0)

            def bucket(v, off_b):
                rv = slot_c[pl.ds(v * 16, 16)]
                m = (rv >= base) & (rv < base + R)
                cum = plsc.cumsum(jnp.where(m, 1, 0))
                pos = jnp.where(m, off_b + cum - 1, C + 64)
                plsc.store_scatter(slots_l, [pos], rv - base)
                plsc.store_scatter(owners_l, [pos],
                                   core * NP + w0 + v * 16 + lane)
                return off_b + last(cum)
            n_b = lax.fori_loop(0, VR, bucket, 0)
            nbk = (n_b + 63) // 64

            def padlists(k, _):
                pos = k * 16 + lane
                m = pos >= n_b
                tgt = jnp.where(m, pos, C + 64)
                plsc.store_scatter(slots_l, [tgt], R + (pos & 15))
                plsc.store_scatter(owners_l, [tgt],
                                   jnp.full((16,), core * NP, jnp.int32))
                return 0
            lax.fori_loop(n_b // 16, (C + 128) // 16, padlists, 0)
            plsc.subcore_barrier()

            co = (1 - t) * 64

            def sumblk(b, _):
                pltpu.async_copy(
                    gflat.at[owners_l.at[pl.ds(b * 64, 64)]], gbuf,
                    semg).wait()

                def inj(k2, _):
                    rows = k2 * 16 + lane
                    plsc.store_scatter(gbuf, [rows, jnp.full((16,), co,
                                                             jnp.int32)],
                                       jnp.full((16,), 1.0, jnp.float32))
                    return 0
                lax.fori_loop(0, 4, inj, 0)
                pltpu.sync_copy(gbuf,
                                sums_sh.at[slots_l.at[pl.ds(b * 64, 64)]],
                                add=True)
                return 0
            lax.fori_loop(0, nbk, sumblk, 0)

            delta = (sidx - core) * NP

            def shiftgids(k, _):
                pos = k * 16 + lane
                g = owners_l[pl.ds(k * 16, 16)] + delta
                owners_l[pl.ds(k * 16, 16)] = jnp.where(
                    pos < n_b, g, sidx * NP + N + (pos & 255))
                return 0
            lax.fori_loop(0, (C + 128) // 16, shiftgids, 0)
            plsc.subcore_barrier()

            def outblk(b, _):
                pltpu.async_copy(
                    sums_sh.at[slots_l.at[pl.ds(b * 64, 64)]], gbuf,
                    semg).wait()
                pltpu.sync_copy(gbuf,
                                spp_out.at[owners_l.at[pl.ds(b * 64, 64)]])
                return 0
            lax.fori_loop(0, nbk, outblk, 0)
            plsc.subcore_barrier()
            return 0
        lax.fori_loop(0, NRANGE, range_body, 0)
        return 0

    lax.fori_loop(0, 2, scale_body, 0)


_CH3 = 3136


@functools.partial(
    pl.kernel, mesh=_mesh, compiler_params=_sc_params,
    out_type=jax.ShapeDtypeStruct((N, 256), jnp.float32),
    scratch_types=[
        pltpu.VMEM((_CH3,), jnp.int32),
        pltpu.VMEM((64, 256), jnp.float32),
        pltpu.SemaphoreType.DMA,
    ])
def _gather_kernel(proj, inv, out, idxb, rowb, sem):
    core = lax.axis_index("c")
    s = lax.axis_index("s")
    wid = s * _NC + core
    b0 = wid * _CH3

    @pl.when(wid < 31)
    def _():
        pltpu.sync_copy(inv.at[pl.ds(b0, _CH3)], idxb)

        def blk(b, _):
            pltpu.async_copy(proj.at[idxb.at[pl.ds(b * 64, 64)]], rowb,
                             sem).wait()
            pltpu.sync_copy(rowb, out.at[pl.ds(b0 + b * 64, 64), :])
            return 0
        lax.fori_loop(0, _CH3 // 64, blk, 0)

    @pl.when(wid == 31)
    def _():
        rem = N - 31 * _CH3
        pltpu.sync_copy(inv.at[pl.ds(b0, rem)], idxb.at[pl.ds(0, rem)])

        def blk(b, _):
            pltpu.async_copy(proj.at[idxb.at[pl.ds(b * 64, 64)]], rowb,
                             sem).wait()
            pltpu.sync_copy(rowb, out.at[pl.ds(b0 + b * 64, 64), :])
            return 0
        lax.fori_loop(0, rem // 64, blk, 0)
        tb = rem // 64 * 64
        pltpu.async_copy(proj.at[idxb.at[pl.ds(tb, 32)]],
                         rowb.at[pl.ds(0, 32)], sem).wait()
        pltpu.sync_copy(rowb.at[pl.ds(0, 32)],
                        out.at[pl.ds(b0 + tb, 32), :])


def kernel(input_data, input_coords, input_coords_inv, W_red, b_red,
           W_fcl0, b_fcl0, W_fcl1, b_fcl1, W_fcl2, b_fcl2, W_fcl3, b_fcl3,
           W_fcs0, b_fcs0, W_fcs1, b_fcs1, W_fcs2, b_fcs2, W_fcs3, b_fcs3,
           W_fc, W_out, W_lo1, W_lo2, b_lo2):
    w_cat = jnp.concatenate([W_fcl0, W_fcl1, W_fcl2, W_fcl3], axis=1)
    x_p = jnp.concatenate(
        [input_data, jnp.zeros((NP - N, input_data.shape[1]), jnp.float32)])
    rf, g2 = _phase_a(x_p, W_red, b_red, w_cat)
    gflat = g2.reshape(2 * NP, 128)

    pad_coords = jnp.concatenate(
        [jnp.full((NP - N, 1), 500, jnp.int32),
         jnp.zeros((NP - N, 3), jnp.int32)], axis=1)
    cc = jnp.concatenate([input_coords, pad_coords])
    coords_t = jnp.concatenate([
        (cc[:, 0] << 9) | cc[:, 1],
        (cc[:, 2] << 9) | cc[:, 3]])
    params_c = jnp.array([240, 120, 80, 60, 8192, 4096, 2731, 2048,
                          0, 0, 0, 0, 0, 0, 0, 0], jnp.int32)
    ar = jnp.arange(6400, dtype=jnp.int32)
    iota_c = ar
    neg1_c = jnp.full((6400,), -1, jnp.int32)
    zero_c = jnp.zeros((6400,), jnp.int32)
    dmy_c = T + (ar & 2047)
    zrow_c = jnp.zeros((128, 128), jnp.float32)

    spp = _segment_kernel(coords_t, params_c, iota_c, neg1_c, zero_c,
                          dmy_c, zrow_c, gflat)

    b_fcl_cat = jnp.concatenate([b_fcl0, b_fcl1, b_fcl2, b_fcl3])
    w_fcs_cat = jnp.concatenate([W_fcs0, W_fcs1, W_fcs2, W_fcs3], axis=1)
    b_fcs_cat = jnp.concatenate([b_fcs0, b_fcs1, b_fcs2, b_fcs3])
    proj = _phase_b(spp, rf, b_fcl_cat, W_fc, w_fcs_cat, b_fcs_cat,
                    W_out, W_lo1, W_lo2, b_lo2)
    return _gather_kernel(proj, input_coords_inv)
